# baseline (speedup 1.0000x reference)
import functools

import jax
import jax.numpy as jnp
from jax import lax
from jax.experimental import pallas as pl
from jax.experimental.pallas import tpu as pltpu
from jax.experimental.pallas import tpu_sc as plsc

D = 128
DP = D // 2
N_NODES = 10000
DEG = 32

NC = 2
NS = 16
NW = NC * NS

CH = 120
_GRAN = NW * CH

S = 1
NSL = N_NODES // S
BES = NSL * DEG
_B_RAW_S = BES + NSL
B_SLAB = -(-_B_RAW_S // (2 * _GRAN)) * (2 * _GRAN)
BPW = B_SLAB // NW
NCHUNK = BPW // CH
NPAIR = NCHUNK // 2

BN = 80
GRID = NSL // BN
UOFF = BES // BN

LAST_NEIGH = BES - (NW - 1) * BPW
U_LEN = B_SLAB - BES


def _sc_gather_body(table_hbm, neigh_hbm, nodes_hbm, out_hbm, idx_v, buf0,
                    buf1, gsem0, gsem1, ssem0, ssem1):
    c = lax.axis_index("c")
    s = lax.axis_index("s")
    wid = s * NC + c
    base = wid * BPW

    @pl.when(wid < NW - 1)
    def _():
        pltpu.sync_copy(neigh_hbm.at[pl.ds(base, BPW)], idx_v)

    @pl.when(wid == NW - 1)
    def _():
        pltpu.sync_copy(neigh_hbm.at[pl.ds(base, LAST_NEIGH)],
                        idx_v.at[pl.ds(0, LAST_NEIGH)])
        pltpu.sync_copy(nodes_hbm.at[pl.ds(0, U_LEN)],
                        idx_v.at[pl.ds(LAST_NEIGH, U_LEN)])

    def gather(j, buf, sem):
        return pltpu.make_async_copy(
            table_hbm.at[idx_v.at[pl.ds(j * CH, CH)]], buf, sem)

    def scat(j, buf, sem):
        return pltpu.make_async_copy(
            buf, out_hbm.at[pl.ds(base + j * CH, CH)], sem)

    gather(0, buf0, gsem0).start()

    def pair_body(p, carry):
        a = 2 * p
        gather(a, buf0, gsem0).wait()

        @pl.when(p > 0)
        def _():
            scat(a - 1, buf1, ssem1).wait()

        gather(a + 1, buf1, gsem1).start()
        scat(a, buf0, ssem0).start()
        gather(a + 1, buf1, gsem1).wait()
        scat(a, buf0, ssem0).wait()

        @pl.when(p + 1 < NPAIR)
        def _():
            gather(a + 2, buf0, gsem0).start()

        scat(a + 1, buf1, ssem1).start()
        return carry

    lax.fori_loop(0, NPAIR, pair_body, 0)
    scat(NCHUNK - 1, buf1, ssem1).wait()


def _make_sc_gather():
    return functools.partial(
        pl.kernel,
        mesh=plsc.VectorSubcoreMesh(core_axis_name="c", subcore_axis_name="s"),
        out_type=jax.ShapeDtypeStruct((B_SLAB, D), jnp.float32),
        scratch_types=[
            pltpu.VMEM((BPW,), jnp.int32),
            pltpu.VMEM((CH, D), jnp.float32),
            pltpu.VMEM((CH, D), jnp.float32),
            pltpu.SemaphoreType.DMA,
            pltpu.SemaphoreType.DMA,
            pltpu.SemaphoreType.DMA,
            pltpu.SemaphoreType.DMA,
        ],
    )(_sc_gather_body)


def _tc_mlp_body(e_ref, u_ref, w1a_ref, w1b_ref, b1_ref, w2_ref, b2_ref,
                 w3b_ref, out_ref):
    e = e_ref[...]
    u = u_ref[...]
    h1 = jnp.dot(e.astype(jnp.bfloat16), w1a_ref[...],
                 preferred_element_type=jnp.float32)
    hu = jnp.dot(u.astype(jnp.bfloat16), w1b_ref[...],
                 preferred_element_type=jnp.float32)
    hu = hu + b1_ref[...]
    h1 = h1.reshape(BN, DEG, D) + hu[:, None, :]
    h1 = jnp.maximum(h1, 0.0).reshape(BN * DEG, D)
    h2 = jnp.dot(h1.astype(jnp.bfloat16), w2_ref[...],
                 preferred_element_type=jnp.float32)
    h2 = jnp.maximum(h2 + b2_ref[...], 0.0)
    lb = jnp.dot(h2.astype(jnp.bfloat16), w3b_ref[...],
                 preferred_element_type=jnp.float32)
    l3 = lb.reshape(BN, DEG, D)
    m = jnp.max(l3, axis=1, keepdims=True)
    p = jnp.exp(l3 - m)
    att = p / jnp.sum(p, axis=1, keepdims=True)
    out_ref[...] = jnp.sum(e.reshape(BN, DEG, D) * att, axis=1)


_tc_mlp = pl.pallas_call(
    _tc_mlp_body,
    grid=(GRID,),
    in_specs=[
        pl.BlockSpec((BN * DEG, D), lambda i: (i, 0)),
        pl.BlockSpec((BN, D), lambda i: (UOFF + i, 0)),
        pl.BlockSpec((D, D), lambda i: (0, 0)),
        pl.BlockSpec((D, D), lambda i: (0, 0)),
        pl.BlockSpec((1, D), lambda i: (0, 0)),
        pl.BlockSpec((D, D), lambda i: (0, 0)),
        pl.BlockSpec((1, D), lambda i: (0, 0)),
        pl.BlockSpec((D, D), lambda i: (0, 0)),
    ],
    out_specs=pl.BlockSpec((BN, D), lambda i: (i, 0)),
    out_shape=jax.ShapeDtypeStruct((NSL, D), jnp.float32),
    compiler_params=pltpu.CompilerParams(
        dimension_semantics=("arbitrary",)),
)


def kernel(nodes, neigh_idx, u2e, W1, b1, W2, b2, W3, b3):
    w1a = W1[:D].astype(jnp.bfloat16)
    w1b = W1[D:].astype(jnp.bfloat16)
    w2 = W2.astype(jnp.bfloat16)
    b1r = b1.reshape(1, D)
    b2r = b2.reshape(1, D)
    w3b = jnp.tile(W3, (1, D)).astype(jnp.bfloat16)
    nodes_pad = jnp.concatenate([
        nodes, jnp.zeros((U_LEN - NSL,), jnp.int32)])
    g = _make_sc_gather()(u2e, neigh_idx.reshape(-1), nodes_pad)
    return _tc_mlp(g, g, w1a, w1b, b1r, w2, b2r, w3b)


# speedup vs baseline: 1.0406x; 1.0406x over previous
import functools

import jax
import jax.numpy as jnp
from jax import lax
from jax.experimental import pallas as pl
from jax.experimental.pallas import tpu as pltpu
from jax.experimental.pallas import tpu_sc as plsc

D = 128
DP = D // 2
N_NODES = 10000
DEG = 32

NC = 2
NS = 16
NW = NC * NS

CH = 120
_GRAN = NW * CH

S = 1
NSL = N_NODES // S
BES = NSL * DEG
_B_RAW_S = BES + NSL
B_SLAB = -(-_B_RAW_S // (2 * _GRAN)) * (2 * _GRAN)
BPW = B_SLAB // NW
NCHUNK = BPW // CH
NPAIR = NCHUNK // 2

BN = 80
GRID = NSL // BN
UOFF = BES // BN

LAST_NEIGH = BES - (NW - 1) * BPW
U_LEN = B_SLAB - BES


def _sc_gather_body(table_hbm, neigh_hbm, nodes_hbm, out_hbm, idx_v, buf0,
                    buf1, gsem0, gsem1, ssem0, ssem1):
    c = lax.axis_index("c")
    s = lax.axis_index("s")
    wid = s * NC + c
    base = wid * BPW

    @pl.when(wid < NW - 1)
    def _():
        pltpu.sync_copy(neigh_hbm.at[pl.ds(base, BPW)], idx_v)

    @pl.when(wid == NW - 1)
    def _():
        pltpu.sync_copy(neigh_hbm.at[pl.ds(base, LAST_NEIGH)],
                        idx_v.at[pl.ds(0, LAST_NEIGH)])
        pltpu.sync_copy(nodes_hbm.at[pl.ds(0, U_LEN)],
                        idx_v.at[pl.ds(LAST_NEIGH, U_LEN)])

    def gather(j, buf, sem):
        return pltpu.make_async_copy(
            table_hbm.at[idx_v.at[pl.ds(j * CH, CH)]], buf, sem)

    def scat(j, buf, sem):
        return pltpu.make_async_copy(
            buf, out_hbm.at[pl.ds(base + j * CH, CH)], sem)

    gather(0, buf0, gsem0).start()

    def pair_body(p, carry):
        a = 2 * p
        gather(a, buf0, gsem0).wait()

        @pl.when(p > 0)
        def _():
            scat(a - 1, buf1, ssem1).wait()

        gather(a + 1, buf1, gsem1).start()
        scat(a, buf0, ssem0).start()
        gather(a + 1, buf1, gsem1).wait()
        scat(a, buf0, ssem0).wait()

        @pl.when(p + 1 < NPAIR)
        def _():
            gather(a + 2, buf0, gsem0).start()

        scat(a + 1, buf1, ssem1).start()
        return carry

    lax.fori_loop(0, NPAIR, pair_body, 0)
    scat(NCHUNK - 1, buf1, ssem1).wait()


def _make_sc_gather():
    return functools.partial(
        pl.kernel,
        mesh=plsc.VectorSubcoreMesh(core_axis_name="c", subcore_axis_name="s"),
        out_type=jax.ShapeDtypeStruct((B_SLAB, D), jnp.float32),
        scratch_types=[
            pltpu.VMEM((BPW,), jnp.int32),
            pltpu.VMEM((CH, D), jnp.float32),
            pltpu.VMEM((CH, D), jnp.float32),
            pltpu.SemaphoreType.DMA,
            pltpu.SemaphoreType.DMA,
            pltpu.SemaphoreType.DMA,
            pltpu.SemaphoreType.DMA,
        ],
    )(_sc_gather_body)


def _tc_mlp_body(e_ref, u_ref, w1a_ref, w1b_ref, b1_ref, w2_ref, b2_ref,
                 w3t_ref, out_ref):
    e = e_ref[...]
    u = u_ref[...]
    h1 = jnp.dot(e.astype(jnp.bfloat16), w1a_ref[...],
                 preferred_element_type=jnp.float32)
    hu = jnp.dot(u.astype(jnp.bfloat16), w1b_ref[...],
                 preferred_element_type=jnp.float32)
    hu = hu + b1_ref[...]
    h1 = h1.reshape(BN, DEG, D) + hu[:, None, :]
    h1 = jnp.maximum(h1, 0.0).reshape(BN * DEG, D)
    h2 = jnp.dot(h1.astype(jnp.bfloat16), w2_ref[...],
                 preferred_element_type=jnp.float32)
    h2 = jnp.maximum(h2 + b2_ref[...], 0.0)
    h23 = h2.reshape(BN, DEG, D)
    logits = jnp.sum(h23 * w3t_ref[...][None], axis=2, keepdims=True)
    m = jnp.max(logits, axis=1, keepdims=True)
    p = jnp.exp(logits - m)
    att = p / jnp.sum(p, axis=1, keepdims=True)
    out_ref[...] = jnp.sum(e.reshape(BN, DEG, D) * att, axis=1)


_tc_mlp = pl.pallas_call(
    _tc_mlp_body,
    grid=(GRID,),
    in_specs=[
        pl.BlockSpec((BN * DEG, D), lambda i: (i, 0)),
        pl.BlockSpec((BN, D), lambda i: (UOFF + i, 0)),
        pl.BlockSpec((D, D), lambda i: (0, 0)),
        pl.BlockSpec((D, D), lambda i: (0, 0)),
        pl.BlockSpec((1, D), lambda i: (0, 0)),
        pl.BlockSpec((D, D), lambda i: (0, 0)),
        pl.BlockSpec((1, D), lambda i: (0, 0)),
        pl.BlockSpec((1, D), lambda i: (0, 0)),
    ],
    out_specs=pl.BlockSpec((BN, D), lambda i: (i, 0)),
    out_shape=jax.ShapeDtypeStruct((NSL, D), jnp.float32),
    compiler_params=pltpu.CompilerParams(
        dimension_semantics=("arbitrary",)),
)


def kernel(nodes, neigh_idx, u2e, W1, b1, W2, b2, W3, b3):
    w1a = W1[:D].astype(jnp.bfloat16)
    w1b = W1[D:].astype(jnp.bfloat16)
    w2 = W2.astype(jnp.bfloat16)
    b1r = b1.reshape(1, D)
    b2r = b2.reshape(1, D)
    w3t = W3.reshape(1, D)
    nodes_pad = jnp.concatenate([
        nodes, jnp.zeros((U_LEN - NSL,), jnp.int32)])
    g = _make_sc_gather()(u2e, neigh_idx.reshape(-1), nodes_pad)
    return _tc_mlp(g, g, w1a, w1b, b1r, w2, b2r, w3t)
